# Initial kernel scaffold; baseline (speedup 1.0000x reference)
import functools

import jax
import jax.numpy as jnp
from jax import lax
from jax.experimental import pallas as pl
from jax.experimental.pallas import tpu as pltpu
from jax.experimental.pallas import tpu_sc as plsc

CHUNK = 128


@functools.lru_cache(maxsize=None)
def _make_gather(n_workers: int, n_chunks: int, embed: int, n_cores: int):
    b_per_w = n_chunks * CHUNK
    b_flat = n_workers * b_per_w
    mesh = plsc.VectorSubcoreMesh(core_axis_name="c", subcore_axis_name="s")

    @functools.partial(
        pl.kernel,
        mesh=mesh,
        out_type=jax.ShapeDtypeStruct((b_flat, embed), jnp.float32),
        scratch_types=[
            pltpu.VMEM((n_chunks, CHUNK), jnp.int32),
            pltpu.VMEM((CHUNK, embed), jnp.float32),
            pltpu.SemaphoreType.DMA,
        ],
    )
    def emb_kernel(idx_hbm, table_hbm, out_hbm, idx_v, rows_v, sem):
        wid = lax.axis_index("s") * n_cores + lax.axis_index("c")
        base = wid * b_per_w
        pltpu.sync_copy(idx_hbm.at[wid], idx_v)

        def body(g, carry):
            pltpu.async_copy(table_hbm.at[idx_v.at[g]], rows_v, sem).wait()
            pltpu.sync_copy(rows_v, out_hbm.at[pl.ds(base + g * CHUNK, CHUNK)])
            return carry

        lax.fori_loop(0, n_chunks, body, 0)

    return emb_kernel


def kernel(token_ids, table):
    b, l = token_ids.shape
    vocab, embed = table.shape
    flat = token_ids.reshape(-1).astype(jnp.int32)
    info = plsc.get_sparse_core_info()
    n_workers = info.num_cores * info.num_subcores
    grain = n_workers * CHUNK
    b_flat = flat.shape[0]
    pad = (-b_flat) % grain
    if pad:
        flat = jnp.pad(flat, (0, pad))
    n_chunks = flat.shape[0] // grain
    idx3 = flat.reshape(n_workers, n_chunks, CHUNK)
    out = _make_gather(n_workers, n_chunks, embed, info.num_cores)(idx3, table)
    if pad:
        out = out[:b_flat]
    return out.reshape(b, l, embed)


# speedup vs baseline: 1.6857x; 1.6857x over previous
import functools

import jax
import jax.numpy as jnp
from jax import lax
from jax.experimental import pallas as pl
from jax.experimental.pallas import tpu as pltpu
from jax.experimental.pallas import tpu_sc as plsc

CHUNK = 128


@functools.lru_cache(maxsize=None)
def _make_gather(n_workers: int, n_chunks: int, embed: int, n_cores: int):
    b_per_w = n_chunks * CHUNK
    b_flat = n_workers * b_per_w
    mesh = plsc.VectorSubcoreMesh(core_axis_name="c", subcore_axis_name="s")

    @functools.partial(
        pl.kernel,
        mesh=mesh,
        out_type=jax.ShapeDtypeStruct((b_flat, embed), jnp.float32),
        scratch_types=[
            pltpu.VMEM((n_chunks, CHUNK), jnp.int32),
            pltpu.VMEM((CHUNK, embed), jnp.float32),
            pltpu.SemaphoreType.DMA,
        ],
        compiler_params=pltpu.CompilerParams(use_tc_tiling_on_sc=False),
    )
    def emb_kernel(idx_hbm, table_hbm, out_hbm, idx_v, rows_v, sem):
        wid = lax.axis_index("s") * n_cores + lax.axis_index("c")
        base = wid * b_per_w
        pltpu.sync_copy(idx_hbm.at[wid], idx_v)

        def body(g, carry):
            pltpu.async_copy(table_hbm.at[idx_v.at[g]], rows_v, sem).wait()
            pltpu.sync_copy(rows_v, out_hbm.at[pl.ds(base + g * CHUNK, CHUNK)])
            return carry

        lax.fori_loop(0, n_chunks, body, 0)

    return emb_kernel


def kernel(token_ids, table):
    b, l = token_ids.shape
    vocab, embed = table.shape
    flat = token_ids.reshape(-1).astype(jnp.int32)
    info = plsc.get_sparse_core_info()
    n_workers = info.num_cores * info.num_subcores
    grain = n_workers * CHUNK
    b_flat = flat.shape[0]
    pad = (-b_flat) % grain
    if pad:
        flat = jnp.pad(flat, (0, pad))
    n_chunks = flat.shape[0] // grain
    idx3 = flat.reshape(n_workers, n_chunks, CHUNK)
    out = _make_gather(n_workers, n_chunks, embed, info.num_cores)(idx3, table)
    if pad:
        out = out[:b_flat]
    return out.reshape(b, l, embed)


# speedup vs baseline: 1.8767x; 1.1133x over previous
import functools

import jax
import jax.numpy as jnp
from jax import lax
from jax.experimental import pallas as pl
from jax.experimental.pallas import tpu as pltpu
from jax.experimental.pallas import tpu_sc as plsc

CHUNK = 128
K = 5


@functools.lru_cache(maxsize=None)
def _make_gather(n_workers: int, n_blocks: int, embed: int, n_cores: int):
    bpb = K * CHUNK
    b_per_w = n_blocks * bpb
    b_flat = n_workers * b_per_w
    n_chunks = n_blocks * K
    mesh = plsc.VectorSubcoreMesh(core_axis_name="c", subcore_axis_name="s")

    @functools.partial(
        pl.kernel,
        mesh=mesh,
        out_type=jax.ShapeDtypeStruct((b_flat, embed), jnp.float32),
        scratch_types=[
            pltpu.VMEM((n_chunks, CHUNK), jnp.int32),
            pltpu.VMEM((2, bpb, embed), jnp.float32),
            pltpu.SemaphoreType.DMA,
            pltpu.SemaphoreType.DMA,
            pltpu.SemaphoreType.DMA,
            pltpu.SemaphoreType.DMA,
        ],
        compiler_params=pltpu.CompilerParams(use_tc_tiling_on_sc=False),
    )
    def emb_kernel(idx_hbm, table_hbm, out_hbm, idx_v, rows_v, g0, g1, w0, w1):
        gsem = (g0, g1)
        wsem = (w0, w1)
        wid = lax.axis_index("s") * n_cores + lax.axis_index("c")
        base = wid * b_per_w
        pltpu.sync_copy(idx_hbm.at[wid], idx_v)

        def g_copy(blk, b, j):
            return pltpu.make_async_copy(
                table_hbm.at[idx_v.at[blk * K + j]],
                rows_v.at[b].at[pl.ds(j * CHUNK, CHUNK)],
                gsem[b],
            )

        def w_copy(blk, b):
            return pltpu.make_async_copy(
                rows_v.at[b],
                out_hbm.at[pl.ds(base + blk * bpb, bpb)],
                wsem[b],
            )

        def start_g(blk, b):
            for j in range(K):
                g_copy(blk, b, j).start()

        def wait_g(blk, b):
            for j in range(K):
                g_copy(blk, b, j).wait()

        start_g(0, 0)
        start_g(1, 1)
        wait_g(0, 0)
        w_copy(0, 0).start()

        def body(i, carry):
            blk = 2 * i + 1
            w_copy(blk - 1, 0).wait()
            start_g(blk + 1, 0)
            wait_g(blk, 1)
            w_copy(blk, 1).start()
            blk2 = 2 * i + 2
            w_copy(blk2 - 1, 1).wait()
            start_g(blk2 + 1, 1)
            wait_g(blk2, 0)
            w_copy(blk2, 0).start()
            return carry

        lax.fori_loop(0, (n_blocks - 2) // 2, body, 0)

        last = n_blocks - 1
        w_copy(last - 1, 0).wait()
        wait_g(last, 1)
        w_copy(last, 1).start()
        w_copy(last, 1).wait()

    return emb_kernel


def kernel(token_ids, table):
    b, l = token_ids.shape
    vocab, embed = table.shape
    flat = token_ids.reshape(-1).astype(jnp.int32)
    info = plsc.get_sparse_core_info()
    n_workers = info.num_cores * info.num_subcores
    grain = n_workers * CHUNK * K * 2
    b_flat = flat.shape[0]
    pad = (-b_flat) % grain
    if pad:
        flat = jnp.pad(flat, (0, pad))
    n_blocks = flat.shape[0] // (n_workers * CHUNK * K)
    idx3 = flat.reshape(n_workers, n_blocks * K, CHUNK)
    out = _make_gather(n_workers, n_blocks, embed, info.num_cores)(idx3, table)
    if pad:
        out = out[:b_flat]
    return out.reshape(b, l, embed)


# speedup vs baseline: 2.6625x; 1.4188x over previous
import functools

import jax
import jax.numpy as jnp
from jax import lax
from jax.experimental import pallas as pl
from jax.experimental.pallas import tpu as pltpu
from jax.experimental.pallas import tpu_sc as plsc

CHUNK = 128


@functools.lru_cache(maxsize=None)
def _make_gather_t(n_workers: int, seq: int, n_cores: int):
    n_blocks = seq * 4
    mesh = plsc.VectorSubcoreMesh(core_axis_name="c", subcore_axis_name="s")

    @functools.partial(
        pl.kernel,
        mesh=mesh,
        out_type=jax.ShapeDtypeStruct((seq, 8, n_workers * 4, 8, 128),
                                      jnp.float32),
        scratch_types=[
            pltpu.VMEM((n_blocks, CHUNK), jnp.int32),
            pltpu.VMEM((n_blocks, CHUNK), jnp.int32),
            pltpu.VMEM((2, CHUNK, 128), jnp.float32),
            pltpu.VMEM((2, 8, 1, 8, 128), jnp.float32),
            pltpu.SemaphoreType.DMA,
            pltpu.SemaphoreType.DMA,
            pltpu.SemaphoreType.DMA,
            pltpu.SemaphoreType.DMA,
            pltpu.SemaphoreType.DMA,
        ],
        compiler_params=pltpu.CompilerParams(use_tc_tiling_on_sc=True,
                                             needs_layout_passes=False,
                                             disable_bounds_checks=True),
    )
    def emb_kernel(idx_hbm, table_hbm, out_hbm, idx_v, pair_v, g_v, t_v,
                   g0, g1, w0, w1, isem):
        gsem = (g0, g1)
        wsem = (w0, w1)
        wid = lax.axis_index("s") * n_cores + lax.axis_index("c")

        def i_copy(r):
            return pltpu.make_async_copy(
                idx_hbm.at[r >> 2, pl.ds(wid * 512 + (r & 3) * CHUNK, CHUNK)],
                idx_v.at[r],
                isem,
            )

        lax.fori_loop(0, n_blocks, lambda r, c: (i_copy(r).start(), c)[1], 0)
        lax.fori_loop(0, n_blocks, lambda r, c: (i_copy(r).wait(), c)[1], 0)

        iot = lax.iota(jnp.int32, 16)
        cd_v = [(iot + d) & 15 for d in range(16)]

        @plsc.parallel_loop(0, n_blocks, unroll=2)
        def pair_body(r):
            for c in range(8):
                v = idx_v[r, pl.ds(c * 16, 16)]
                pair_v[r, pl.ds(c * 16, 16)] = v >> 1

        def g_copy(j, p):
            return pltpu.make_async_copy(
                table_hbm.at[pair_v.at[j]],
                g_v.at[p],
                gsem[p],
            )

        def w_copy(j, p):
            return pltpu.make_async_copy(
                t_v.at[p],
                out_hbm.at[j >> 2, :, pl.ds(wid * 4 + (j & 3), 1)],
                wsem[p],
            )

        def transpose(j, p):
            gb = g_v.at[p]
            tb = t_v.at[p]
            zero = jnp.full((16,), 0, jnp.int32)

            @plsc.parallel_loop(0, 32, unroll=2)
            def body(s):
                t0 = (s >> 2) * 16
                e0 = (s & 3) * 16
                rows = iot + t0
                par = (idx_v[j, pl.ds(t0, 16)] & 1) << 6
                base = par + e0
                for d in range(16):
                    vals = plsc.load_gather(gb, [rows, cd_v[d] + base])
                    plsc.store_scatter(
                        tb,
                        [(cd_v[d] >> 3) + (e0 >> 3), zero, cd_v[d] & 7, rows],
                        vals)

        def section(j, p, first, last):
            if not last:
                g_copy(j + 1, 1 - p).start()
            g_copy(j, p).wait()
            if not first:
                w_copy(j - 2, p).wait()
            transpose(j, p)
            w_copy(j, p).start()

        g_copy(0, 0).start()
        section(0, 0, True, False)
        section(1, 1, True, False)

        def body(i, carry):
            section(2 * i + 2, 0, False, False)
            section(2 * i + 3, 1, False, False)
            return carry

        lax.fori_loop(0, (n_blocks - 4) // 2, body, 0)

        section(n_blocks - 2, 0, False, False)
        section(n_blocks - 1, 1, False, True)
        w_copy(n_blocks - 2, 0).wait()
        w_copy(n_blocks - 1, 1).wait()

    return emb_kernel


def kernel(token_ids, table):
    b, l = token_ids.shape
    vocab, embed = table.shape
    info = plsc.get_sparse_core_info()
    n_workers = info.num_cores * info.num_subcores
    assert embed == 64 and b == n_workers * 512 and l >= 2 and l % 2 == 0 \
        and vocab % 2 == 0, "kernel specialized for the problem shapes"
    idx = token_ids.astype(jnp.int32).T
    table2 = table.reshape(vocab // 2, 2 * embed)
    out5 = _make_gather_t(n_workers, l, info.num_cores)(idx, table2)
    out = jnp.transpose(out5, (2, 4, 0, 1, 3)).reshape(b, l, embed)
    return out


# speedup vs baseline: 2.8851x; 1.0836x over previous
import functools

import jax
import jax.numpy as jnp
from jax import lax
from jax.experimental import pallas as pl
from jax.experimental.pallas import tpu as pltpu
from jax.experimental.pallas import tpu_sc as plsc

CHUNK = 128
TBLK = 256


@functools.lru_cache(maxsize=None)
def _make_gather_t(n_workers: int, seq: int, n_cores: int):
    n_blocks = seq * 2
    n_rows = n_blocks * (TBLK // CHUNK)
    mesh = plsc.VectorSubcoreMesh(core_axis_name="c", subcore_axis_name="s")

    @functools.partial(
        pl.kernel,
        mesh=mesh,
        out_type=jax.ShapeDtypeStruct((seq, 8, n_workers * 4, 8, 128),
                                      jnp.float32),
        scratch_types=[
            pltpu.VMEM((n_rows, CHUNK), jnp.int32),
            pltpu.VMEM((2, TBLK, 64), jnp.float32),
            pltpu.VMEM((2, 8, 2, 8, 128), jnp.float32),
            pltpu.SemaphoreType.DMA,
            pltpu.SemaphoreType.DMA,
            pltpu.SemaphoreType.DMA,
            pltpu.SemaphoreType.DMA,
            pltpu.SemaphoreType.DMA,
        ],
        compiler_params=pltpu.CompilerParams(use_tc_tiling_on_sc=False,
                                             needs_layout_passes=False,
                                             disable_bounds_checks=True),
    )
    def emb_kernel(idx_hbm, table_hbm, out_hbm, idx_v, g_v, t_v,
                   g0, g1, w0, w1, isem):
        gsem = (g0, g1)
        wsem = (w0, w1)
        wid = lax.axis_index("s") * n_cores + lax.axis_index("c")

        def i_copy(r):
            return pltpu.make_async_copy(
                idx_hbm.at[r >> 2, pl.ds(wid * 512 + (r & 3) * CHUNK, CHUNK)],
                idx_v.at[r],
                isem,
            )

        def i_start(r, carry):
            i_copy(r).start()
            return carry

        def i_wait(r, carry):
            i_copy(r).wait()
            return carry

        lax.fori_loop(0, n_rows, i_start, 0)
        lax.fori_loop(0, n_rows, i_wait, 0)

        iot = lax.iota(jnp.int32, 16)
        cd_v = [(iot + d) & 15 for d in range(16)]

        def g_copy(j, p, c):
            return pltpu.make_async_copy(
                table_hbm.at[idx_v.at[2 * j + c]],
                g_v.at[p, pl.ds(c * CHUNK, CHUNK)],
                gsem[p],
            )

        def w_copy(j, p):
            return pltpu.make_async_copy(
                t_v.at[p],
                out_hbm.at[j // 2, :, pl.ds(wid * 4 + (j % 2) * 2, 2)],
                wsem[p],
            )

        def start_g(j, p):
            g_copy(j, p, 0).start()
            g_copy(j, p, 1).start()

        def wait_g(j, p):
            g_copy(j, p, 0).wait()
            g_copy(j, p, 1).wait()

        def transpose(p):
            gb = g_v.at[p]
            tb = t_v.at[p]

            @plsc.parallel_loop(0, 64, unroll=4)
            def body(s):
                t0 = (s >> 2) * 16
                e0 = (s & 3) * 16
                rows = iot + t0
                bt2 = jnp.full((16,), t0 >> 7, jnp.int32)
                bi = iot + (t0 & 127)
                et0 = e0 >> 3
                for d in range(16):
                    vals = plsc.load_gather(gb, [rows, cd_v[d] + e0])
                    plsc.store_scatter(
                        tb,
                        [(cd_v[d] >> 3) + et0, bt2, cd_v[d] & 7, bi],
                        vals)

        def section(j, p, first, last):
            if not last:
                start_g(j + 1, 1 - p)
            wait_g(j, p)
            if not first:
                w_copy(j - 2, p).wait()
            transpose(p)
            w_copy(j, p).start()

        start_g(0, 0)
        section(0, 0, True, False)
        section(1, 1, True, False)

        def body(i, carry):
            section(2 * i + 2, 0, False, False)
            section(2 * i + 3, 1, False, False)
            return carry

        lax.fori_loop(0, (n_blocks - 4) // 2, body, 0)

        section(n_blocks - 2, 0, False, False)
        section(n_blocks - 1, 1, False, True)
        w_copy(n_blocks - 2, 0).wait()
        w_copy(n_blocks - 1, 1).wait()

    return emb_kernel


def kernel(token_ids, table):
    b, l = token_ids.shape
    vocab, embed = table.shape
    info = plsc.get_sparse_core_info()
    n_workers = info.num_cores * info.num_subcores
    assert embed == 64 and b == n_workers * 512 and l >= 4 and l % 2 == 0, (
        "kernel specialized for the problem shapes")
    idx = token_ids.astype(jnp.int32).T
    out5 = _make_gather_t(n_workers, l, info.num_cores)(idx, table)
    out = jnp.transpose(out5, (2, 4, 0, 1, 3)).reshape(b, l, embed)
    return out


# speedup vs baseline: 3.0105x; 1.0435x over previous
import functools

import jax
import jax.numpy as jnp
from jax import lax
from jax.experimental import pallas as pl
from jax.experimental.pallas import tpu as pltpu
from jax.experimental.pallas import tpu_sc as plsc

CHUNK = 128
TBLK = 256


@functools.lru_cache(maxsize=None)
def _make_gather_t(n_workers: int, seq: int, n_cores: int):
    n_blocks = seq * 2
    n_rows = n_blocks * (TBLK // CHUNK)
    mesh = plsc.VectorSubcoreMesh(core_axis_name="c", subcore_axis_name="s")

    @functools.partial(
        pl.kernel,
        mesh=mesh,
        out_type=jax.ShapeDtypeStruct((seq, 8, n_workers * 4, 8, 128),
                                      jnp.float32),
        scratch_types=[
            pltpu.VMEM((n_rows, CHUNK), jnp.int32),
            pltpu.VMEM((2, TBLK, 64), jnp.float32),
            pltpu.VMEM((2, 8, 2, 8, 128), jnp.float32),
            pltpu.SemaphoreType.DMA,
            pltpu.SemaphoreType.DMA,
            pltpu.SemaphoreType.DMA,
            pltpu.SemaphoreType.DMA,
            pltpu.SemaphoreType.DMA,
        ],
        compiler_params=pltpu.CompilerParams(use_tc_tiling_on_sc=False,
                                             needs_layout_passes=False,
                                             disable_bounds_checks=True),
    )
    def emb_kernel(idx_hbm, table_hbm, out_hbm, idx_v, g_v, t_v,
                   g0, g1, w0, w1, isem):
        gsem = (g0, g1)
        wsem = (w0, w1)
        wid = lax.axis_index("s") * n_cores + lax.axis_index("c")

        def i_copy(r):
            return pltpu.make_async_copy(
                idx_hbm.at[r >> 2, pl.ds(wid * 512 + (r & 3) * CHUNK, CHUNK)],
                idx_v.at[r],
                isem,
            )

        def i_start(r, carry):
            i_copy(r).start()
            return carry

        def i_wait(r, carry):
            i_copy(r).wait()
            return carry

        lax.fori_loop(0, n_rows, i_start, 0)
        lax.fori_loop(0, n_rows, i_wait, 0)

        iot = lax.iota(jnp.int32, 16)
        cd_v = [(iot + d) & 15 for d in range(16)]

        def g_copy(j, p, c):
            return pltpu.make_async_copy(
                table_hbm.at[idx_v.at[2 * j + c]],
                g_v.at[p, pl.ds(c * CHUNK, CHUNK)],
                gsem[p],
            )

        def w_copy(j, p):
            return pltpu.make_async_copy(
                t_v.at[p],
                out_hbm.at[j // 2, :, pl.ds(wid * 4 + (j % 2) * 2, 2)],
                wsem[p],
            )

        def start_g(j, p):
            g_copy(j, p, 0).start()
            g_copy(j, p, 1).start()

        def wait_g(j, p):
            g_copy(j, p, 0).wait()
            g_copy(j, p, 1).wait()

        def transpose(p):
            gb = g_v.at[p]
            tb = t_v.at[p]

            @plsc.parallel_loop(0, 64, unroll=2)
            def body(s):
                t0 = (s >> 2) * 16
                e0 = (s & 3) * 16
                rows = iot + t0
                bt2 = jnp.full((16,), t0 >> 7, jnp.int32)
                bi = iot + (t0 & 127)
                et0 = e0 >> 3
                for d in range(16):
                    vals = plsc.load_gather(gb, [rows, cd_v[d] + e0])
                    plsc.store_scatter(
                        tb,
                        [(cd_v[d] >> 3) + et0, bt2, cd_v[d] & 7, bi],
                        vals)

        def section(j, p, first, last):
            if not last:
                start_g(j + 1, 1 - p)
            wait_g(j, p)
            if not first:
                w_copy(j - 2, p).wait()
            transpose(p)
            w_copy(j, p).start()

        start_g(0, 0)
        section(0, 0, True, False)
        section(1, 1, True, False)

        def body(i, carry):
            section(2 * i + 2, 0, False, False)
            section(2 * i + 3, 1, False, False)
            return carry

        lax.fori_loop(0, (n_blocks - 4) // 2, body, 0)

        section(n_blocks - 2, 0, False, False)
        section(n_blocks - 1, 1, False, True)
        w_copy(n_blocks - 2, 0).wait()
        w_copy(n_blocks - 1, 1).wait()

    return emb_kernel


def kernel(token_ids, table):
    b, l = token_ids.shape
    vocab, embed = table.shape
    info = plsc.get_sparse_core_info()
    n_workers = info.num_cores * info.num_subcores
    assert embed == 64 and b == n_workers * 512 and l >= 4 and l % 2 == 0, (
        "kernel specialized for the problem shapes")
    idx = token_ids.astype(jnp.int32).T
    out5 = _make_gather_t(n_workers, l, info.num_cores)(idx, table)
    out = jnp.transpose(out5, (2, 4, 0, 1, 3)).reshape(b, l, embed)
    return out
